# baseline (speedup 1.0000x reference)
import functools

import jax
import jax.numpy as jnp
from jax import lax
from jax.experimental import pallas as pl
from jax.experimental.pallas import tpu as pltpu
from jax.experimental.pallas import tpu_sc as plsc

_NC = 2
_NS = 16


def _sc_mesh():
    return plsc.VectorSubcoreMesh(core_axis_name="c", subcore_axis_name="s",
                                  num_cores=_NC, num_subcores=_NS)


def _per_tile_rows(n_rows, s, copy_fn):
    r0 = (n_rows // _NS) // 8 * 8
    last = n_rows - r0 * (_NS - 1)

    @pl.when(s < _NS - 1)
    def _():
        copy_fn(pl.multiple_of(s * r0, 8), r0)

    @pl.when(s == _NS - 1)
    def _():
        copy_fn((_NS - 1) * r0, last)


def _sc_degree(dst_blk, zeros8, ones8, E):
    N = zeros8.shape[0]
    K = ones8.shape[0]
    per_tile = E // (_NC * _NS)
    nchunk = per_tile // K

    @functools.partial(
        pl.kernel,
        out_type=jax.ShapeDtypeStruct((_NC, N, 8), jnp.float32),
        mesh=_sc_mesh(),
        compiler_params=pltpu.CompilerParams(use_tc_tiling_on_sc=False),
        scratch_types=[
            pltpu.VMEM((per_tile // K, K), jnp.int32),
            pltpu.VMEM((K, 8), jnp.float32),
            pltpu.VMEM_SHARED((N, 8), jnp.float32),
            pltpu.SemaphoreType.DMA,
        ],
    )
    def deg_kernel(dst_hbm, z_hbm, ones_hbm, out_hbm, dstb, onesbuf, deg_sh,
                   sem):
        c = lax.axis_index("c")
        s = lax.axis_index("s")
        w = c * _NS + s
        pltpu.sync_copy(dst_hbm.at[pl.ds(w * nchunk, nchunk)], dstb)
        pltpu.sync_copy(ones_hbm, onesbuf)
        _per_tile_rows(N, s, lambda r, m: pltpu.sync_copy(
            z_hbm.at[pl.ds(r, m)], deg_sh.at[pl.ds(r, m)]))
        plsc.subcore_barrier()

        def body(j, carry):
            pltpu.async_copy(onesbuf, deg_sh.at[dstb.at[j]], sem, add=True)
            return carry

        lax.fori_loop(0, nchunk, body, 0)

        def drain(j, carry):
            pltpu.make_async_copy(onesbuf, deg_sh.at[dstb.at[j]], sem).wait()
            return carry

        lax.fori_loop(0, nchunk, drain, 0)
        plsc.subcore_barrier()
        _per_tile_rows(N, s, lambda r, m: pltpu.sync_copy(
            deg_sh.at[pl.ds(r, m)], out_hbm.at[c, pl.ds(r, m)]))

    return deg_kernel(dst_blk, zeros8, ones8)


def _sc_scatter(g2n, src_blk, dst_blk, E, K, NB):
    N = g2n.shape[0] // 2
    D2 = g2n.shape[1]
    per_tile = E // _NS
    nchunk = per_tile // K
    n_it = nchunk // NB

    @functools.partial(
        pl.kernel,
        out_type=jax.ShapeDtypeStruct((_NC, N, D2), jnp.float32),
        mesh=_sc_mesh(),
        compiler_params=pltpu.CompilerParams(use_tc_tiling_on_sc=False),
        scratch_types=[
            pltpu.VMEM((nchunk, K), jnp.int32),
            pltpu.VMEM((nchunk, K), jnp.int32),
            [pltpu.VMEM((K, D2), jnp.float32)] * NB,
            pltpu.VMEM_SHARED((N, D2), jnp.float32),
            [pltpu.SemaphoreType.DMA] * NB,
            [pltpu.SemaphoreType.DMA] * NB,
        ],
    )
    def scat_kernel(g_hbm, src_hbm, dst_hbm, out_hbm,
                    srcb, dstb, rows, acc, sem_g, sem_s):
        c = lax.axis_index("c")
        s = lax.axis_index("s")
        pltpu.sync_copy(src_hbm.at[pl.ds((c * _NS + s) * nchunk, nchunk)],
                        srcb)
        pltpu.sync_copy(dst_hbm.at[pl.ds(s * nchunk, nchunk)], dstb)
        for b in range(NB - 1):
            pltpu.async_copy(g_hbm.at[srcb.at[b]], rows[b], sem_g[b])
        _per_tile_rows(N, s, lambda r, m: pltpu.sync_copy(
            g_hbm.at[pl.ds(c * N + r, m)], acc.at[pl.ds(r, m)]))
        plsc.subcore_barrier()

        def body(i, carry):
            for b in range(NB):
                j = NB * i + b
                bn = (b + NB - 1) % NB
                pltpu.make_async_copy(g_hbm.at[srcb.at[j]], rows[b],
                                      sem_g[b]).wait()
                pltpu.async_copy(rows[b], acc.at[dstb.at[j]], sem_s[b],
                                 add=True)

                def wait_prev():
                    pltpu.make_async_copy(rows[bn], acc.at[dstb.at[j - 1]],
                                          sem_s[bn]).wait()

                def issue_next():
                    pltpu.async_copy(g_hbm.at[srcb.at[j + NB - 1]], rows[bn],
                                     sem_g[bn])

                if b == 0:
                    pl.when(i > 0)(wait_prev)
                    issue_next()
                else:
                    def recycle():
                        wait_prev()
                        issue_next()

                    pl.when(i + 1 < n_it)(recycle)
            return carry

        lax.fori_loop(0, n_it, body, 0)
        for j in range(nchunk - NB, nchunk):
            pltpu.make_async_copy(rows[j % NB], acc.at[dstb.at[j]],
                                  sem_s[j % NB]).wait()
        plsc.subcore_barrier()
        _per_tile_rows(N, s, lambda r, m: pltpu.sync_copy(
            acc.at[pl.ds(r, m)], out_hbm.at[c, pl.ds(r, m)]))

    return scat_kernel(g2n, src_blk, dst_blk)


def _tc_layer1(x, W1, p0, p1, blk):
    N, D = x.shape
    H = W1.shape[1]
    Hh = H // 2

    def body(x_ref, w_ref, p0_ref, p1_ref, g_ref, dv_ref):
        dv = lax.rsqrt(p0_ref[...] + p1_ref[...] + 1.0)
        h = jnp.dot(x_ref[...], w_ref[...],
                    preferred_element_type=jnp.float32)
        g_ref[0] = h[:, :Hh] * dv
        g_ref[1] = h[:, Hh:] * dv
        dv_ref[...] = dv

    return pl.pallas_call(
        body,
        grid=(N // blk,),
        in_specs=[
            pl.BlockSpec((blk, D), lambda i: (i, 0)),
            pl.BlockSpec((D, H), lambda i: (0, 0)),
            pl.BlockSpec((blk, 1), lambda i: (i, 0)),
            pl.BlockSpec((blk, 1), lambda i: (i, 0)),
        ],
        out_specs=[
            pl.BlockSpec((2, blk, Hh), lambda i: (0, i, 0)),
            pl.BlockSpec((blk, 1), lambda i: (i, 0)),
        ],
        out_shape=[
            jax.ShapeDtypeStruct((2, N, Hh), jnp.float32),
            jax.ShapeDtypeStruct((N, 1), jnp.float32),
        ],
    )(x, W1, p0, p1)


def _tc_layer2(s1, dinv, b1, W2, blk):
    N = s1.shape[1]
    H = 2 * s1.shape[2]
    O = W2.shape[1]
    Oh = O // 2

    def body(s_ref, dv_ref, b_ref, w_ref, g_ref):
        dv = dv_ref[...]
        h = jnp.concatenate([s_ref[0], s_ref[1]], axis=1) * dv + b_ref[...]
        h = jnp.maximum(h, 0.0)
        g = jnp.dot(h, w_ref[...], preferred_element_type=jnp.float32) * dv
        g_ref[0] = g[:, :Oh]
        g_ref[1] = g[:, Oh:]

    return pl.pallas_call(
        body,
        grid=(N // blk,),
        in_specs=[
            pl.BlockSpec((2, blk, H // 2), lambda i: (0, i, 0)),
            pl.BlockSpec((blk, 1), lambda i: (i, 0)),
            pl.BlockSpec((1, H), lambda i: (0, 0)),
            pl.BlockSpec((H, O), lambda i: (0, 0)),
        ],
        out_specs=pl.BlockSpec((2, blk, Oh), lambda i: (0, i, 0)),
        out_shape=jax.ShapeDtypeStruct((2, N, Oh), jnp.float32),
    )(s1, dinv, b1, W2)


def _tc_layer3(s2, dinv, b2, blk):
    N = s2.shape[1]
    O = 2 * s2.shape[2]

    def body(s_ref, dv_ref, b_ref, o_ref):
        o = (jnp.concatenate([s_ref[0], s_ref[1]], axis=1) * dv_ref[...]
             + b_ref[...])
        m = jnp.max(o, axis=1, keepdims=True)
        e = jnp.exp(o - m)
        lse = jnp.log(jnp.sum(e, axis=1, keepdims=True))
        o_ref[...] = o - m - lse

    return pl.pallas_call(
        body,
        grid=(N // blk,),
        in_specs=[
            pl.BlockSpec((2, blk, O // 2), lambda i: (0, i, 0)),
            pl.BlockSpec((blk, 1), lambda i: (i, 0)),
            pl.BlockSpec((1, O), lambda i: (0, 0)),
        ],
        out_specs=pl.BlockSpec((blk, O), lambda i: (i, 0)),
        out_shape=jax.ShapeDtypeStruct((N, O), jnp.float32),
    )(s2, dinv, b2)


def kernel(x, edge_index, W1, b1, W2, b2):
    N = x.shape[0]
    E = edge_index.shape[1]
    K = 20
    src = edge_index[0].astype(jnp.int32)
    dst = edge_index[1].astype(jnp.int32)
    src_blk = jnp.concatenate([src, src + N]).reshape(-1, K)
    dst_blk = dst.reshape(-1, K)

    zeros8 = jnp.zeros((N, 8), jnp.float32)
    ones8 = jnp.ones((K, 8), jnp.float32)
    parts = _sc_degree(dst_blk, zeros8, ones8, E)
    p0 = parts[0, :, 0:1]
    p1 = parts[1, :, 0:1]

    blk = 1000
    g1, dinv = _tc_layer1(x, W1, p0, p1, blk)
    s1 = _sc_scatter(g1.reshape(2 * N, -1), src_blk, dst_blk, E, K, 10)
    g2 = _tc_layer2(s1, dinv, b1.reshape(1, -1), W2, blk)
    s2 = _sc_scatter(g2.reshape(2 * N, -1), src_blk, dst_blk, E, K, 10)
    return _tc_layer3(s2, dinv, b2.reshape(1, -1), blk)


# speedup vs baseline: 1.1513x; 1.1513x over previous
import functools

import jax
import jax.numpy as jnp
from jax import lax
from jax.experimental import pallas as pl
from jax.experimental.pallas import tpu as pltpu
from jax.experimental.pallas import tpu_sc as plsc

_NC = 2
_NS = 16


def _sc_mesh():
    return plsc.VectorSubcoreMesh(core_axis_name="c", subcore_axis_name="s",
                                  num_cores=_NC, num_subcores=_NS)


def _per_tile_rows(n_rows, s, copy_fn):
    r0 = (n_rows // _NS) // 8 * 8
    last = n_rows - r0 * (_NS - 1)

    @pl.when(s < _NS - 1)
    def _():
        copy_fn(pl.multiple_of(s * r0, 8), r0)

    @pl.when(s == _NS - 1)
    def _():
        copy_fn((_NS - 1) * r0, last)


def _sc_degree(dst_blk, zeros8, ones8, E):
    N = zeros8.shape[0]
    K = ones8.shape[0]
    per_tile = E // (_NC * _NS)
    nchunk = per_tile // K

    @functools.partial(
        pl.kernel,
        out_type=jax.ShapeDtypeStruct((_NC, N, 8), jnp.float32),
        mesh=_sc_mesh(),
        compiler_params=pltpu.CompilerParams(use_tc_tiling_on_sc=False),
        scratch_types=[
            pltpu.VMEM((per_tile // K, K), jnp.int32),
            pltpu.VMEM((K, 8), jnp.float32),
            pltpu.VMEM_SHARED((N, 8), jnp.float32),
            pltpu.SemaphoreType.DMA,
        ],
    )
    def deg_kernel(dst_hbm, z_hbm, ones_hbm, out_hbm, dstb, onesbuf, deg_sh,
                   sem):
        c = lax.axis_index("c")
        s = lax.axis_index("s")
        w = c * _NS + s
        pltpu.sync_copy(dst_hbm.at[pl.ds(w * nchunk, nchunk)], dstb)
        pltpu.sync_copy(ones_hbm, onesbuf)
        _per_tile_rows(N, s, lambda r, m: pltpu.sync_copy(
            z_hbm.at[pl.ds(r, m)], deg_sh.at[pl.ds(r, m)]))
        plsc.subcore_barrier()

        def body(j, carry):
            pltpu.async_copy(onesbuf, deg_sh.at[dstb.at[j]], sem, add=True)
            return carry

        lax.fori_loop(0, nchunk, body, 0)

        def drain(j, carry):
            pltpu.make_async_copy(onesbuf, deg_sh.at[dstb.at[j]], sem).wait()
            return carry

        lax.fori_loop(0, nchunk, drain, 0)
        plsc.subcore_barrier()
        _per_tile_rows(N, s, lambda r, m: pltpu.sync_copy(
            deg_sh.at[pl.ds(r, m)], out_hbm.at[c, pl.ds(r, m)]))

    return deg_kernel(dst_blk, zeros8, ones8)


def _sc_scatter(g2n, src_blk, dst_blk, E, K, NB):
    N = g2n.shape[0] // 2
    D2 = g2n.shape[1]
    per_tile = E // _NS
    nchunk = per_tile // K
    n_it = nchunk // NB

    @functools.partial(
        pl.kernel,
        out_type=jax.ShapeDtypeStruct((_NC, N, D2), jnp.float32),
        mesh=_sc_mesh(),
        compiler_params=pltpu.CompilerParams(use_tc_tiling_on_sc=False),
        scratch_types=[
            pltpu.VMEM((nchunk, K), jnp.int32),
            pltpu.VMEM((nchunk, K), jnp.int32),
            [pltpu.VMEM((K, D2), jnp.float32)] * NB,
            pltpu.VMEM_SHARED((N, D2), jnp.float32),
            [pltpu.SemaphoreType.DMA] * NB,
            [pltpu.SemaphoreType.DMA] * NB,
        ],
    )
    def scat_kernel(g_hbm, src_hbm, dst_hbm, out_hbm,
                    srcb, dstb, rows, acc, sem_g, sem_s):
        c = lax.axis_index("c")
        s = lax.axis_index("s")
        pltpu.sync_copy(src_hbm.at[pl.ds((c * _NS + s) * nchunk, nchunk)],
                        srcb)
        pltpu.sync_copy(dst_hbm.at[pl.ds(s * nchunk, nchunk)], dstb)
        for b in range(NB - 1):
            pltpu.async_copy(g_hbm.at[srcb.at[b]], rows[b], sem_g[b])
        _per_tile_rows(N, s, lambda r, m: pltpu.sync_copy(
            g_hbm.at[pl.ds(c * N + r, m)], acc.at[pl.ds(r, m)]))
        plsc.subcore_barrier()

        def body(i, carry):
            for b in range(NB):
                j = NB * i + b
                bn = (b + NB - 1) % NB
                pltpu.make_async_copy(g_hbm.at[srcb.at[j]], rows[b],
                                      sem_g[b]).wait()
                pltpu.async_copy(rows[b], acc.at[dstb.at[j]], sem_s[b],
                                 add=True)

                def wait_prev():
                    pltpu.make_async_copy(rows[bn], acc.at[dstb.at[j - 1]],
                                          sem_s[bn]).wait()

                def issue_next():
                    pltpu.async_copy(g_hbm.at[srcb.at[j + NB - 1]], rows[bn],
                                     sem_g[bn])

                if b == 0:
                    pl.when(i > 0)(wait_prev)
                    issue_next()
                else:
                    def recycle():
                        wait_prev()
                        issue_next()

                    pl.when(i + 1 < n_it)(recycle)
            return carry

        lax.fori_loop(0, n_it, body, 0)
        for j in range(nchunk - NB, nchunk):
            pltpu.make_async_copy(rows[j % NB], acc.at[dstb.at[j]],
                                  sem_s[j % NB]).wait()
        plsc.subcore_barrier()
        _per_tile_rows(N, s, lambda r, m: pltpu.sync_copy(
            acc.at[pl.ds(r, m)], out_hbm.at[c, pl.ds(r, m)]))

    return scat_kernel(g2n, src_blk, dst_blk)


def _tc_layer1(x, W1, p0, p1, blk):
    N, D = x.shape
    H = W1.shape[1]
    Hh = H // 2

    def body(x_ref, w_ref, p0_ref, p1_ref, g_ref, dv_ref):
        dv = lax.rsqrt(p0_ref[...] + p1_ref[...] + 1.0)
        h = jnp.dot(x_ref[...], w_ref[...],
                    preferred_element_type=jnp.float32)
        g_ref[0] = h[:, :Hh] * dv
        g_ref[1] = h[:, Hh:] * dv
        dv_ref[...] = dv

    return pl.pallas_call(
        body,
        grid=(N // blk,),
        in_specs=[
            pl.BlockSpec((blk, D), lambda i: (i, 0)),
            pl.BlockSpec((D, H), lambda i: (0, 0)),
            pl.BlockSpec((blk, 1), lambda i: (i, 0)),
            pl.BlockSpec((blk, 1), lambda i: (i, 0)),
        ],
        out_specs=[
            pl.BlockSpec((2, blk, Hh), lambda i: (0, i, 0)),
            pl.BlockSpec((blk, 1), lambda i: (i, 0)),
        ],
        out_shape=[
            jax.ShapeDtypeStruct((2, N, Hh), jnp.float32),
            jax.ShapeDtypeStruct((N, 1), jnp.float32),
        ],
    )(x, W1, p0, p1)


def _tc_layer2(s1, dinv, b1, W2, blk):
    N = s1.shape[1]
    H = 2 * s1.shape[2]
    O = W2.shape[1]
    Oh = O // 2

    def body(s_ref, dv_ref, b_ref, w_ref, g_ref):
        dv = dv_ref[...]
        h = jnp.concatenate([s_ref[0], s_ref[1]], axis=1) * dv + b_ref[...]
        h = jnp.maximum(h, 0.0)
        g = jnp.dot(h, w_ref[...], preferred_element_type=jnp.float32) * dv
        g_ref[0] = g[:, :Oh]
        g_ref[1] = g[:, Oh:]

    return pl.pallas_call(
        body,
        grid=(N // blk,),
        in_specs=[
            pl.BlockSpec((2, blk, H // 2), lambda i: (0, i, 0)),
            pl.BlockSpec((blk, 1), lambda i: (i, 0)),
            pl.BlockSpec((1, H), lambda i: (0, 0)),
            pl.BlockSpec((H, O), lambda i: (0, 0)),
        ],
        out_specs=pl.BlockSpec((2, blk, Oh), lambda i: (0, i, 0)),
        out_shape=jax.ShapeDtypeStruct((2, N, Oh), jnp.float32),
    )(s1, dinv, b1, W2)


def _tc_layer3(s2, dinv, b2, blk):
    N = s2.shape[1]
    O = 2 * s2.shape[2]

    def body(s_ref, dv_ref, b_ref, o_ref):
        o = (jnp.concatenate([s_ref[0], s_ref[1]], axis=1) * dv_ref[...]
             + b_ref[...])
        m = jnp.max(o, axis=1, keepdims=True)
        e = jnp.exp(o - m)
        lse = jnp.log(jnp.sum(e, axis=1, keepdims=True))
        o_ref[...] = o - m - lse

    return pl.pallas_call(
        body,
        grid=(N // blk,),
        in_specs=[
            pl.BlockSpec((2, blk, O // 2), lambda i: (0, i, 0)),
            pl.BlockSpec((blk, 1), lambda i: (i, 0)),
            pl.BlockSpec((1, O), lambda i: (0, 0)),
        ],
        out_specs=pl.BlockSpec((blk, O), lambda i: (i, 0)),
        out_shape=jax.ShapeDtypeStruct((N, O), jnp.float32),
    )(s2, dinv, b2)


def kernel(x, edge_index, W1, b1, W2, b2):
    N = x.shape[0]
    E = edge_index.shape[1]
    K = 40
    src = edge_index[0].astype(jnp.int32)
    dst = edge_index[1].astype(jnp.int32)
    src_blk = jnp.concatenate([src, src + N]).reshape(-1, K)
    dst_blk = dst.reshape(-1, K)

    zeros8 = jnp.zeros((N, 8), jnp.float32)
    ones8 = jnp.ones((K, 8), jnp.float32)
    parts = _sc_degree(dst_blk, zeros8, ones8, E)
    p0 = parts[0, :, 0:1]
    p1 = parts[1, :, 0:1]

    blk = 1000
    g1, dinv = _tc_layer1(x, W1, p0, p1, blk)
    s1 = _sc_scatter(g1.reshape(2 * N, -1), src_blk, dst_blk, E, K, 5)
    g2 = _tc_layer2(s1, dinv, b1.reshape(1, -1), W2, blk)
    K2 = 100
    src_blk2 = src_blk.reshape(-1, K2)
    dst_blk2 = dst_blk.reshape(-1, K2)
    s2 = _sc_scatter(g2.reshape(2 * N, -1), src_blk2, dst_blk2, E, K2, 10)
    return _tc_layer3(s2, dinv, b2.reshape(1, -1), blk)
